# baseline (speedup 1.0000x reference)
import functools

import jax
import jax.numpy as jnp
import numpy as np
from jax.experimental import pallas as pl
from jax.experimental.pallas import tpu as pltpu

N_BASIS = 9
TILE = 16384


def _acos(x):
    a = jnp.abs(x)
    poly = (1.5707963050 + a * (-0.2145988016 + a * (0.0889789874 + a * (
        -0.0501743046 + a * (0.0308918810 + a * (-0.0170881256 + a * (
            0.0066700901 + a * -0.0012624911)))))))
    r = jnp.sqrt(jnp.maximum(1.0 - a, 0.0)) * poly
    return jnp.where(x >= 0.0, r, np.pi - r)


def _dot(a, b):
    return jax.lax.dot_general(
        a, b, (((1,), (0,)), ((), ())),
        preferred_element_type=jnp.float32,
    )


def _dotb(wref, x):
    return jax.lax.dot_general(
        wref[...], x.astype(jnp.bfloat16), (((1,), (0,)), ((), ())),
        preferred_element_type=jnp.float32,
    )


def _core_kernel(p_ref, ld_ref, li_ref, cam_ref,
                 wg0_ref, bg0_ref, wg1_ref, bg1_ref,
                 wg2f_ref, bg2f_ref, w2col_ref, wg1b_ref, wg0b_ref,
                 wm0p_ref, wm0f_ref, bm0_ref, wm1_ref, bm1_ref,
                 ws0_ref, bs0_ref, ws1_ref, bs1_ref, ssum_ref,
                 out_ref):
    p = p_ref[...]
    ld = ld_ref[...]
    li = li_ref[...]
    cam = cam_ref[...]

    z0 = _dot(wg0_ref[...], p) + bg0_ref[...]
    y0 = 100.0 * z0
    h0 = (jnp.maximum(y0, 0.0) + jnp.log1p(jnp.exp(-jnp.abs(y0)))) * 0.01
    z1 = _dot(wg1_ref[...], h0) + bg1_ref[...]
    y1 = 100.0 * z1
    h1 = (jnp.maximum(y1, 0.0) + jnp.log1p(jnp.exp(-jnp.abs(y1)))) * 0.01
    feats = _dotb(wg2f_ref, h1) + bg2f_ref[...]

    d1 = w2col_ref[...] * jax.nn.sigmoid(y1)
    d0 = _dot(wg1b_ref[...], d1) * jax.nn.sigmoid(y0)
    grad = _dot(wg0b_ref[...], d0)

    gnorm = jnp.sqrt(jnp.sum(grad * grad, axis=0, keepdims=True))
    normals = grad / (gnorm + 1e-6)

    view = cam - p
    vnorm = jnp.sqrt(jnp.sum(view * view, axis=0, keepdims=True))
    view = view / (vnorm + 1e-6)

    half = ld + view
    hnorm = jnp.sqrt(jnp.sum(half * half, axis=0, keepdims=True))
    half = half / (hnorm + 1e-6)

    eps = 1e-6
    cos_th = jnp.clip(jnp.sum(half * normals, axis=0, keepdims=True),
                      -1.0 + eps, 1.0 - eps)
    cos_td = jnp.clip(jnp.sum(half * view, axis=0, keepdims=True),
                      -1.0 + eps, 1.0 - eps)
    thd = _acos(jnp.concatenate([cos_th, cos_td], axis=0))

    hm = jax.nn.relu(_dotb(wm0p_ref, p) +
                     _dotb(wm0f_ref, feats) + bm0_ref[...])
    m1 = _dotb(wm1_ref, hm) + bm1_ref[...]
    albedo = jax.nn.sigmoid(m1[0:3, :])
    spec_coeff = m1[8:17, :]

    hs = jax.nn.relu(_dot(ws0_ref[...], thd) + bs0_ref[...])
    bas = jax.nn.relu(_dotb(ws1_ref, hs) + bs1_ref[...])
    coeff3 = jnp.concatenate([spec_coeff] * 3, axis=0)
    spec_ref = _dotb(ssum_ref, bas * coeff3)

    brdf = albedo + spec_ref
    shading = jax.nn.relu(jnp.sum(normals * ld, axis=0, keepdims=True))
    out_ref[...] = jnp.clip(shading * brdf * li, 0.0, 1.0)


@functools.partial(jax.jit, static_argnames=("interpret",))
def _run(pT, *ops, interpret=False):
    n = pT.shape[1]
    grid = (n // TILE,)

    def col_block(shape):
        return pl.BlockSpec(shape, lambda i: (0, i))

    def whole(shape):
        return pl.BlockSpec(shape, lambda i: (0,) * len(shape))

    in_specs = [col_block((3, TILE))] + [whole(o.shape) for o in ops]
    return pl.pallas_call(
        _core_kernel,
        grid=grid,
        in_specs=in_specs,
        out_specs=col_block((3, TILE)),
        out_shape=jax.ShapeDtypeStruct((3, n), jnp.float32),
        compiler_params=pltpu.CompilerParams(
            dimension_semantics=(pltpu.PARALLEL,),
            vmem_limit_bytes=100 * 1024 * 1024,
        ),
        interpret=interpret,
    )(pT, *ops)


def kernel(vertsparam, light_dir, light_int, cam_loc,
           Wg0, bg0, Wg1, bg1, Wg2, bg2,
           Wd0, bd0, Wd1, bd1, Wc0, bc0, Wc1, bc1,
           Ws0, bs0, Ws1, bs1):
    f32 = jnp.float32
    pT = vertsparam.T
    ld = light_dir / (jnp.linalg.norm(light_dir) + 1e-6)
    ld = ld[:, None]
    li = light_int[:, None]
    cam = cam_loc[:, None]

    Wg0T = Wg0.T
    bg0s = bg0
    Wg1T = Wg1.T
    bg1s = bg1
    Wg2fT = Wg2[:, 1:].T.astype(jnp.bfloat16)
    bg2f = bg2[1:][:, None]
    w2col = Wg2[:, 0][:, None]

    Wm0p = jnp.concatenate([Wd0[:3], Wc0[:3]], axis=1).T.astype(jnp.bfloat16)
    Wm0f = jnp.concatenate([Wd0[3:], Wc0[3:]], axis=1).T.astype(jnp.bfloat16)
    bm0 = jnp.concatenate([bd0, bc0])[:, None]
    d_hid = Wd1.shape[0]
    Wm1T = jnp.concatenate([
        jnp.concatenate([Wd1.T, jnp.zeros((3, d_hid), f32)], axis=1),
        jnp.zeros((5, 2 * d_hid), f32),
        jnp.concatenate([jnp.zeros((9, d_hid), f32), Wc1.T], axis=1),
    ], axis=0).astype(jnp.bfloat16)
    bm1 = jnp.concatenate([bd1, jnp.zeros((5,), f32), bc1])[:, None]

    perm = np.array([j * 3 + k for k in range(3) for j in range(N_BASIS)])
    Ws0T = (1.0 / np.pi) * Ws0.T
    Ws1T = Ws1[:, perm].T.astype(jnp.bfloat16)
    bs1p = bs1[perm][:, None]
    ssum = jnp.asarray(np.kron(np.eye(3), np.ones((1, N_BASIS))),
                       jnp.bfloat16)

    outT = _run(pT, ld, li, cam,
                Wg0T, bg0s[:, None], Wg1T, bg1s[:, None],
                Wg2fT, bg2f, w2col, Wg1, Wg0,
                Wm0p, Wm0f, bm0, Wm1T, bm1,
                Ws0T, bs0[:, None], Ws1T, bs1p, ssum)
    return outT.T


# speedup vs baseline: 1.0182x; 1.0182x over previous
import functools

import jax
import jax.numpy as jnp
import numpy as np
from jax.experimental import pallas as pl
from jax.experimental.pallas import tpu as pltpu

N_BASIS = 9
TILE = 16384


def _acos(x):
    a = jnp.abs(x)
    poly = (1.5707963050 + a * (-0.2145988016 + a * (0.0889789874 + a * (
        -0.0501743046 + a * (0.0308918810 + a * (-0.0170881256 + a * (
            0.0066700901 + a * -0.0012624911)))))))
    r = jnp.sqrt(jnp.maximum(1.0 - a, 0.0)) * poly
    return jnp.where(x >= 0.0, r, np.pi - r)


def _dot(a, b):
    return jax.lax.dot_general(
        a, b, (((1,), (0,)), ((), ())),
        preferred_element_type=jnp.float32,
    )


def _core_kernel(p_ref, ld_ref, li_ref, cam_ref,
                 wg0_ref, bg0_ref, wg1_ref, bg1_ref,
                 wg2f_ref, bg2f_ref, w2col_ref, wg1b_ref, wg0b_ref,
                 wm0p_ref, wm0f_ref, bm0_ref, wm1_ref, bm1_ref,
                 ws0_ref, bs0_ref, ws1_ref, bs1_ref, ssum_ref,
                 out_ref):
    p = p_ref[...]
    ld = ld_ref[...]
    li = li_ref[...]
    cam = cam_ref[...]

    z0 = _dot(wg0_ref[...], p) + bg0_ref[...]
    y0 = 100.0 * z0
    h0 = (jnp.maximum(y0, 0.0) + jnp.log1p(jnp.exp(-jnp.abs(y0)))) * 0.01
    z1 = _dot(wg1_ref[...], h0) + bg1_ref[...]
    y1 = 100.0 * z1
    h1 = (jnp.maximum(y1, 0.0) + jnp.log1p(jnp.exp(-jnp.abs(y1)))) * 0.01
    feats = _dot(wg2f_ref[...], h1) + bg2f_ref[...]

    d1 = w2col_ref[...] * jax.nn.sigmoid(y1)
    d0 = _dot(wg1b_ref[...], d1) * jax.nn.sigmoid(y0)
    grad = _dot(wg0b_ref[...], d0)

    gnorm = jnp.sqrt(jnp.sum(grad * grad, axis=0, keepdims=True))
    normals = grad / (gnorm + 1e-6)

    view = cam - p
    vnorm = jnp.sqrt(jnp.sum(view * view, axis=0, keepdims=True))
    view = view / (vnorm + 1e-6)

    half = ld + view
    hnorm = jnp.sqrt(jnp.sum(half * half, axis=0, keepdims=True))
    half = half / (hnorm + 1e-6)

    eps = 1e-6
    cos_th = jnp.clip(jnp.sum(half * normals, axis=0, keepdims=True),
                      -1.0 + eps, 1.0 - eps)
    cos_td = jnp.clip(jnp.sum(half * view, axis=0, keepdims=True),
                      -1.0 + eps, 1.0 - eps)
    thd = _acos(jnp.concatenate([cos_th, cos_td], axis=0))

    hm = jax.nn.relu(_dot(wm0p_ref[...], p) +
                     _dot(wm0f_ref[...], feats) + bm0_ref[...])
    m1 = _dot(wm1_ref[...], hm) + bm1_ref[...]
    albedo = jax.nn.sigmoid(m1[0:3, :])
    spec_coeff = m1[8:17, :]

    hs = jax.nn.relu(_dot(ws0_ref[...], thd) + bs0_ref[...])
    bas = jax.nn.relu(_dot(ws1_ref[...], hs) + bs1_ref[...])
    coeff3 = jnp.concatenate([spec_coeff] * 3, axis=0)
    spec_ref = _dot(ssum_ref[...], bas * coeff3)

    brdf = albedo + spec_ref
    shading = jax.nn.relu(jnp.sum(normals * ld, axis=0, keepdims=True))
    out_ref[...] = jnp.clip(shading * brdf * li, 0.0, 1.0)


@functools.partial(jax.jit, static_argnames=("interpret",))
def _run(pT, *ops, interpret=False):
    n = pT.shape[1]
    grid = (n // TILE,)

    def col_block(shape):
        return pl.BlockSpec(shape, lambda i: (0, i))

    def whole(shape):
        return pl.BlockSpec(shape, lambda i: (0,) * len(shape))

    in_specs = [col_block((3, TILE))] + [whole(o.shape) for o in ops]
    return pl.pallas_call(
        _core_kernel,
        grid=grid,
        in_specs=in_specs,
        out_specs=col_block((3, TILE)),
        out_shape=jax.ShapeDtypeStruct((3, n), jnp.float32),
        compiler_params=pltpu.CompilerParams(
            dimension_semantics=(pltpu.PARALLEL,),
            vmem_limit_bytes=100 * 1024 * 1024,
        ),
        interpret=interpret,
    )(pT, *ops)


def kernel(vertsparam, light_dir, light_int, cam_loc,
           Wg0, bg0, Wg1, bg1, Wg2, bg2,
           Wd0, bd0, Wd1, bd1, Wc0, bc0, Wc1, bc1,
           Ws0, bs0, Ws1, bs1):
    f32 = jnp.float32
    pT = vertsparam.T
    ld = light_dir / (jnp.linalg.norm(light_dir) + 1e-6)
    ld = ld[:, None]
    li = light_int[:, None]
    cam = cam_loc[:, None]

    Wg0T = Wg0.T
    bg0s = bg0
    Wg1T = Wg1.T
    bg1s = bg1
    Wg2fT = Wg2[:, 1:].T
    bg2f = bg2[1:][:, None]
    w2col = Wg2[:, 0][:, None]

    Wm0p = jnp.concatenate([Wd0[:3], Wc0[:3]], axis=1).T
    Wm0f = jnp.concatenate([Wd0[3:], Wc0[3:]], axis=1).T
    bm0 = jnp.concatenate([bd0, bc0])[:, None]
    d_hid = Wd1.shape[0]
    Wm1T = jnp.concatenate([
        jnp.concatenate([Wd1.T, jnp.zeros((3, d_hid), f32)], axis=1),
        jnp.zeros((5, 2 * d_hid), f32),
        jnp.concatenate([jnp.zeros((9, d_hid), f32), Wc1.T], axis=1),
    ], axis=0)
    bm1 = jnp.concatenate([bd1, jnp.zeros((5,), f32), bc1])[:, None]

    perm = np.array([j * 3 + k for k in range(3) for j in range(N_BASIS)])
    Ws0T = (1.0 / np.pi) * Ws0.T
    Ws1T = Ws1[:, perm].T
    bs1p = bs1[perm][:, None]
    ssum = jnp.asarray(np.kron(np.eye(3), np.ones((1, N_BASIS))), f32)

    outT = _run(pT, ld, li, cam,
                Wg0T, bg0s[:, None], Wg1T, bg1s[:, None],
                Wg2fT, bg2f, w2col, Wg1, Wg0,
                Wm0p, Wm0f, bm0, Wm1T, bm1,
                Ws0T, bs0[:, None], Ws1T, bs1p, ssum)
    return outT.T


# speedup vs baseline: 1.0209x; 1.0027x over previous
import jax
import jax.numpy as jnp
import numpy as np
from jax.experimental import pallas as pl
from jax.experimental.pallas import tpu as pltpu

N_BASIS = 9
TILE = 16384


def _acos(x):
    a = jnp.abs(x)
    poly = (1.5707963050 + a * (-0.2145988016 + a * (0.0889789874 + a * (
        -0.0501743046 + a * (0.0308918810 + a * (-0.0170881256 + a * (
            0.0066700901 + a * -0.0012624911)))))))
    r = jnp.sqrt(jnp.maximum(1.0 - a, 0.0)) * poly
    return jnp.where(x >= 0.0, r, np.pi - r)


def _dot(a, b):
    return jax.lax.dot_general(
        a, b, (((1,), (0,)), ((), ())),
        preferred_element_type=jnp.float32,
    )


def _core_kernel(p_ref, ld_ref, li_ref, cam_ref,
                 wg0_ref, bg0_ref, wg1_ref, bg1_ref,
                 wg2f_ref, bg2f_ref, w2col_ref, wg1b_ref, wg0b_ref,
                 wm0p_ref, wm0f_ref, bm0_ref, wm1_ref, bm1_ref,
                 ws0_ref, bs0_ref, ws1_ref, bs1_ref, ssum_ref,
                 out_ref):
    p = p_ref[...]
    ld = ld_ref[...]
    li = li_ref[...]
    cam = cam_ref[...]

    z0 = _dot(wg0_ref[...], p) + bg0_ref[...]
    y0 = 100.0 * z0
    h0 = (jnp.maximum(y0, 0.0) + jnp.log1p(jnp.exp(-jnp.abs(y0)))) * 0.01
    z1 = _dot(wg1_ref[...], h0) + bg1_ref[...]
    y1 = 100.0 * z1
    h1 = (jnp.maximum(y1, 0.0) + jnp.log1p(jnp.exp(-jnp.abs(y1)))) * 0.01
    feats = _dot(wg2f_ref[...], h1) + bg2f_ref[...]

    d1 = w2col_ref[...] * jax.nn.sigmoid(y1)
    d0 = _dot(wg1b_ref[...], d1) * jax.nn.sigmoid(y0)
    grad = _dot(wg0b_ref[...], d0)

    gnorm = jnp.sqrt(jnp.sum(grad * grad, axis=0, keepdims=True))
    normals = grad / (gnorm + 1e-6)

    view = cam - p
    vnorm = jnp.sqrt(jnp.sum(view * view, axis=0, keepdims=True))
    view = view / (vnorm + 1e-6)

    half = ld + view
    hnorm = jnp.sqrt(jnp.sum(half * half, axis=0, keepdims=True))
    half = half / (hnorm + 1e-6)

    eps = 1e-6
    cos_th = jnp.clip(jnp.sum(half * normals, axis=0, keepdims=True),
                      -1.0 + eps, 1.0 - eps)
    cos_td = jnp.clip(jnp.sum(half * view, axis=0, keepdims=True),
                      -1.0 + eps, 1.0 - eps)
    thd = _acos(jnp.concatenate([cos_th, cos_td], axis=0))

    hm = jax.nn.relu(_dot(wm0p_ref[...], p) +
                     _dot(wm0f_ref[...], feats) + bm0_ref[...])
    m1 = _dot(wm1_ref[...], hm) + bm1_ref[...]
    albedo = jax.nn.sigmoid(m1[0:3, :])
    spec_coeff = m1[8:17, :]

    hs = jax.nn.relu(_dot(ws0_ref[...], thd) + bs0_ref[...])
    bas = jax.nn.relu(_dot(ws1_ref[...], hs) + bs1_ref[...])
    coeff3 = jnp.concatenate([spec_coeff] * 3, axis=0)
    spec_ref = _dot(ssum_ref[...], bas * coeff3)

    brdf = albedo + spec_ref
    shading = jax.nn.relu(jnp.sum(normals * ld, axis=0, keepdims=True))
    out_ref[...] = jnp.clip(shading * brdf * li, 0.0, 1.0)


@jax.jit
def _run(pT, *ops):
    n = pT.shape[1]
    grid = (n // TILE,)

    def col_block(shape):
        return pl.BlockSpec(shape, lambda i: (0, i))

    def whole(shape):
        return pl.BlockSpec(shape, lambda i: (0,) * len(shape))

    in_specs = [col_block((3, TILE))] + [whole(o.shape) for o in ops]
    return pl.pallas_call(
        _core_kernel,
        grid=grid,
        in_specs=in_specs,
        out_specs=col_block((3, TILE)),
        out_shape=jax.ShapeDtypeStruct((3, n), jnp.float32),
        compiler_params=pltpu.CompilerParams(
            dimension_semantics=(pltpu.PARALLEL,),
            vmem_limit_bytes=100 * 1024 * 1024,
        ),
    )(pT, *ops)


def kernel(vertsparam, light_dir, light_int, cam_loc,
           Wg0, bg0, Wg1, bg1, Wg2, bg2,
           Wd0, bd0, Wd1, bd1, Wc0, bc0, Wc1, bc1,
           Ws0, bs0, Ws1, bs1):
    f32 = jnp.float32
    pT = vertsparam.T
    ld = light_dir / (jnp.linalg.norm(light_dir) + 1e-6)
    ld = ld[:, None]
    li = light_int[:, None]
    cam = cam_loc[:, None]

    Wg0T = Wg0.T
    bg0s = bg0
    Wg1T = Wg1.T
    bg1s = bg1
    Wg2fT = Wg2[:, 1:].T
    bg2f = bg2[1:][:, None]
    w2col = Wg2[:, 0][:, None]

    Wm0p = jnp.concatenate([Wd0[:3], Wc0[:3]], axis=1).T
    Wm0f = jnp.concatenate([Wd0[3:], Wc0[3:]], axis=1).T
    bm0 = jnp.concatenate([bd0, bc0])[:, None]
    d_hid = Wd1.shape[0]
    Wm1T = jnp.concatenate([
        jnp.concatenate([Wd1.T, jnp.zeros((3, d_hid), f32)], axis=1),
        jnp.zeros((5, 2 * d_hid), f32),
        jnp.concatenate([jnp.zeros((9, d_hid), f32), Wc1.T], axis=1),
    ], axis=0)
    bm1 = jnp.concatenate([bd1, jnp.zeros((5,), f32), bc1])[:, None]

    perm = np.array([j * 3 + k for k in range(3) for j in range(N_BASIS)])
    Ws0T = (1.0 / np.pi) * Ws0.T
    Ws1T = Ws1[:, perm].T
    bs1p = bs1[perm][:, None]
    ssum = jnp.asarray(np.kron(np.eye(3), np.ones((1, N_BASIS))), f32)

    outT = _run(pT, ld, li, cam,
                Wg0T, bg0s[:, None], Wg1T, bg1s[:, None],
                Wg2fT, bg2f, w2col, Wg1, Wg0,
                Wm0p, Wm0f, bm0, Wm1T, bm1,
                Ws0T, bs0[:, None], Ws1T, bs1p, ssum)
    return outT.T
